# baseline (device time: 24401 ns/iter reference)
import os

import jax
import jax.numpy as jnp
from jax import lax
from jax.experimental import pallas as pl
from jax.experimental.pallas import tpu as pltpu

N_DEV = 4
B = 2
SQ_LOC = 128
D_MODEL = 512
HQ = 16
HQ_LOC = 4
DH = 64
SKV = 128
BLK = 64
GDIM = HQ_LOC * DH
BDR = HQ_LOC * SKV


def kernel(x, Wq, K_ext, V_ext, Wo):
    bf16 = jnp.bfloat16
    f32 = jnp.float32

    def body(x_ref, wq_ref, k_ref, v_ref, wo_ref, out_ref,
             wq_bf, wo_bf, wq_slots, wo_slots, k_bd, v_bd,
             wq_send, wq_recv, wo_send, wo_recv):
        my = lax.axis_index("i")
        left = lax.rem(my + N_DEV - 1, N_DEV)
        right = lax.rem(my + 1, N_DEV)
        opp = lax.rem(my + 2, N_DEV)

        wq_bf[...] = wq_ref[...].astype(bf16)
        wo_bf[...] = wo_ref[...].astype(bf16)

        barrier_sem = pltpu.get_barrier_semaphore()
        for nbr in (left, right, opp):
            pl.semaphore_signal(
                barrier_sem, inc=1,
                device_id=(nbr,), device_id_type=pl.DeviceIdType.MESH,
            )
        pl.semaphore_wait(barrier_sem, N_DEV - 1)

        txs = []

        def push_to(dest):
            for src, slots, ssem, rsem in (
                (wq_bf, wq_slots, wq_send, wq_recv),
                (wo_bf, wo_slots, wo_send, wo_recv),
            ):
                tx = pltpu.make_async_remote_copy(
                    src_ref=src, dst_ref=slots.at[my],
                    send_sem=ssem.at[dest], recv_sem=rsem.at[my],
                    device_id=(dest,), device_id_type=pl.DeviceIdType.MESH,
                )
                tx.start()
                txs.append(tx)

        def wait_from(origin):
            for slots, rsem in ((wq_slots, wq_recv), (wo_slots, wo_recv)):
                rx = pltpu.make_async_remote_copy(
                    src_ref=slots.at[origin], dst_ref=slots.at[origin],
                    send_sem=wq_send.at[origin], recv_sem=rsem.at[origin],
                    device_id=(origin,), device_id_type=pl.DeviceIdType.MESH,
                )
                rx.wait_recv()

        skip_comm = bool(os.environ.get("SKIP_COMM"))
        if not skip_comm:
            push_to(opp)
            push_to(right)
            push_to(left)


        k_bd[...] = jnp.zeros((B, N_DEV, BDR, GDIM), bf16)
        v_bd[...] = jnp.zeros((B, N_DEV, BDR, GDIM), bf16)
        for b in range(B):
            for g in range(N_DEV):
                for hh in range(HQ_LOC):
                    head = g * HQ_LOC + hh
                    r0, c0 = hh * SKV, hh * DH
                    k_bd[b, g, r0:r0 + SKV, c0:c0 + DH] = (
                        k_ref[b, :, head, :].astype(bf16))
                    v_bd[b, g, r0:r0 + SKV, c0:c0 + DH] = (
                        v_ref[b, :, head, :].astype(bf16))

        x2 = (x_ref[...].reshape(B * SQ_LOC, D_MODEL) * 0.125).astype(bf16)

        qi = lax.broadcasted_iota(jnp.int32, (SQ_LOC, HQ_LOC * SKV), 0)
        kj = lax.broadcasted_iota(jnp.int32, (SQ_LOC, HQ_LOC * SKV), 1)
        qb = my * (SQ_LOC // BLK) + qi // BLK
        kb = lax.rem(kj, SKV) // BLK
        mask = (qb == kb) | (kb == 0) | (lax.rem(qb + kb, 3) == 0)
        bias = jnp.where(mask, 0.0, -1e9).astype(f32)

        def compute_group(g, wq_g, wo_g, acc):
            q_g = jax.lax.dot_general(
                x2, wq_g, (((1,), (0,)), ((), ())),
                preferred_element_type=f32,
            ).astype(bf16)
            ctxs = []
            for b in range(B):
                q_b = q_g[b * SQ_LOC:(b + 1) * SQ_LOC, :]
                s = jax.lax.dot_general(
                    q_b, k_bd[b, g], (((1,), (1,)), ((), ())),
                    preferred_element_type=f32,
                )
                w = jnp.exp(s + bias)
                w3 = w.reshape(SQ_LOC, HQ_LOC, SKV)
                w3 = w3 / jnp.sum(w3, axis=-1, keepdims=True)
                w2 = w3.reshape(SQ_LOC, HQ_LOC * SKV).astype(bf16)
                ctxs.append(jax.lax.dot_general(
                    w2, v_bd[b, g], (((1,), (0,)), ((), ())),
                    preferred_element_type=f32,
                ).astype(bf16))
            ctx = jnp.concatenate(ctxs, axis=0)
            return acc + jax.lax.dot_general(
                ctx, wo_g, (((1,), (0,)), ((), ())),
                preferred_element_type=f32,
            )

        acc = jnp.zeros((B * SQ_LOC, D_MODEL), dtype=f32)
        if skip_comm:
            for g in range(N_DEV):
                acc = compute_group(g, wq_bf[...], wo_bf[...], acc)
        else:
            acc = compute_group(my, wq_bf[...], wo_bf[...], acc)
            for origin in (left, right, opp):
                wait_from(origin)
                acc = compute_group(origin, wq_slots[origin],
                                    wo_slots[origin], acc)
            for tx in txs:
                tx.wait_send()

        out_ref[...] = acc.reshape(B, SQ_LOC, D_MODEL)

    return pl.pallas_call(
        body,
        out_shape=jax.ShapeDtypeStruct((B, SQ_LOC, D_MODEL), jnp.float32),
        in_specs=[pl.BlockSpec(memory_space=pltpu.VMEM)] * 5,
        out_specs=pl.BlockSpec(memory_space=pltpu.VMEM),
        scratch_shapes=[
            pltpu.VMEM((D_MODEL, GDIM), bf16),
            pltpu.VMEM((GDIM, D_MODEL), bf16),
            pltpu.VMEM((N_DEV, D_MODEL, GDIM), bf16),
            pltpu.VMEM((N_DEV, GDIM, D_MODEL), bf16),
            pltpu.VMEM((B, N_DEV, BDR, GDIM), bf16),
            pltpu.VMEM((B, N_DEV, BDR, GDIM), bf16),
            pltpu.SemaphoreType.DMA((N_DEV,)),
            pltpu.SemaphoreType.DMA((N_DEV,)),
            pltpu.SemaphoreType.DMA((N_DEV,)),
            pltpu.SemaphoreType.DMA((N_DEV,)),
        ],
        compiler_params=pltpu.CompilerParams(collective_id=0),
    )(x, Wq, K_ext, V_ext, Wo)


# device time: 24315 ns/iter; 1.0035x vs baseline; 1.0035x over previous
import os

import jax
import jax.numpy as jnp
from jax import lax
from jax.experimental import pallas as pl
from jax.experimental.pallas import tpu as pltpu

N_DEV = 4
B = 2
SQ_LOC = 128
D_MODEL = 512
HQ = 16
HQ_LOC = 4
DH = 64
SKV = 128
BLK = 64
GDIM = HQ_LOC * DH
BDR = HQ_LOC * SKV


def kernel(x, Wq, K_ext, V_ext, Wo):
    bf16 = jnp.bfloat16
    f32 = jnp.float32

    def body(x_ref, wq_ref, k_ref, v_ref, wo_ref, out_ref,
             wq_bf, wo_bf, wq_slots, wo_slots, k_bd, v_bd,
             wq_send, wq_recv, wo_send, wo_recv):
        my = lax.axis_index("i")
        left = lax.rem(my + N_DEV - 1, N_DEV)
        right = lax.rem(my + 1, N_DEV)
        opp = lax.rem(my + 2, N_DEV)

        barrier_sem = pltpu.get_barrier_semaphore()
        for nbr in (left, right, opp):
            pl.semaphore_signal(
                barrier_sem, inc=1,
                device_id=(nbr,), device_id_type=pl.DeviceIdType.MESH,
            )
        wq_bf[...] = wq_ref[...].astype(bf16)
        wo_bf[...] = wo_ref[...].astype(bf16)
        pl.semaphore_wait(barrier_sem, N_DEV - 1)

        txs = []

        def push_to(dest):
            for src, slots, ssem, rsem in (
                (wq_bf, wq_slots, wq_send, wq_recv),
                (wo_bf, wo_slots, wo_send, wo_recv),
            ):
                tx = pltpu.make_async_remote_copy(
                    src_ref=src, dst_ref=slots.at[my],
                    send_sem=ssem.at[dest], recv_sem=rsem.at[my],
                    device_id=(dest,), device_id_type=pl.DeviceIdType.MESH,
                )
                tx.start()
                txs.append(tx)

        def wait_from(origin):
            for slots, rsem in ((wq_slots, wq_recv), (wo_slots, wo_recv)):
                rx = pltpu.make_async_remote_copy(
                    src_ref=slots.at[origin], dst_ref=slots.at[origin],
                    send_sem=wq_send.at[origin], recv_sem=rsem.at[origin],
                    device_id=(origin,), device_id_type=pl.DeviceIdType.MESH,
                )
                rx.wait_recv()

        skip_comm = bool(os.environ.get("SKIP_COMM"))
        if not skip_comm:
            push_to(opp)
            push_to(right)
            push_to(left)


        k_bd[...] = jnp.zeros((B, N_DEV, BDR, GDIM), bf16)
        v_bd[...] = jnp.zeros((B, N_DEV, BDR, GDIM), bf16)
        for b in range(B):
            for g in range(N_DEV):
                for hh in range(HQ_LOC):
                    head = g * HQ_LOC + hh
                    r0, c0 = hh * SKV, hh * DH
                    k_bd[b, g, r0:r0 + SKV, c0:c0 + DH] = (
                        k_ref[b, :, head, :].astype(bf16))
                    v_bd[b, g, r0:r0 + SKV, c0:c0 + DH] = (
                        v_ref[b, :, head, :].astype(bf16))

        x2 = (x_ref[...].reshape(B * SQ_LOC, D_MODEL) * 0.125).astype(bf16)

        qi = lax.broadcasted_iota(jnp.int32, (SQ_LOC, HQ_LOC * SKV), 0)
        kj = lax.broadcasted_iota(jnp.int32, (SQ_LOC, HQ_LOC * SKV), 1)
        qb = my * (SQ_LOC // BLK) + qi // BLK
        kb = lax.rem(kj, SKV) // BLK
        mask = (qb == kb) | (kb == 0) | (lax.rem(qb + kb, 3) == 0)
        bias = jnp.where(mask, 0.0, -1e9).astype(f32)

        def compute_group(g, wq_g, wo_g, acc):
            q_g = jax.lax.dot_general(
                x2, wq_g, (((1,), (0,)), ((), ())),
                preferred_element_type=f32,
            ).astype(bf16)
            ctxs = []
            for b in range(B):
                q_b = q_g[b * SQ_LOC:(b + 1) * SQ_LOC, :]
                s = jax.lax.dot_general(
                    q_b, k_bd[b, g], (((1,), (1,)), ((), ())),
                    preferred_element_type=f32,
                )
                w = jnp.exp(s + bias)
                w3 = w.reshape(SQ_LOC, HQ_LOC, SKV)
                w3 = w3 / jnp.sum(w3, axis=-1, keepdims=True)
                w2 = w3.reshape(SQ_LOC, HQ_LOC * SKV).astype(bf16)
                ctxs.append(jax.lax.dot_general(
                    w2, v_bd[b, g], (((1,), (0,)), ((), ())),
                    preferred_element_type=f32,
                ).astype(bf16))
            ctx = jnp.concatenate(ctxs, axis=0)
            return acc + jax.lax.dot_general(
                ctx, wo_g, (((1,), (0,)), ((), ())),
                preferred_element_type=f32,
            )

        acc = jnp.zeros((B * SQ_LOC, D_MODEL), dtype=f32)
        if skip_comm:
            for g in range(N_DEV):
                acc = compute_group(g, wq_bf[...], wo_bf[...], acc)
        else:
            acc = compute_group(my, wq_bf[...], wo_bf[...], acc)
            for origin in (left, right, opp):
                wait_from(origin)
                acc = compute_group(origin, wq_slots[origin],
                                    wo_slots[origin], acc)
            for tx in txs:
                tx.wait_send()

        out_ref[...] = acc.reshape(B, SQ_LOC, D_MODEL)

    return pl.pallas_call(
        body,
        out_shape=jax.ShapeDtypeStruct((B, SQ_LOC, D_MODEL), jnp.float32),
        in_specs=[pl.BlockSpec(memory_space=pltpu.VMEM)] * 5,
        out_specs=pl.BlockSpec(memory_space=pltpu.VMEM),
        scratch_shapes=[
            pltpu.VMEM((D_MODEL, GDIM), bf16),
            pltpu.VMEM((GDIM, D_MODEL), bf16),
            pltpu.VMEM((N_DEV, D_MODEL, GDIM), bf16),
            pltpu.VMEM((N_DEV, GDIM, D_MODEL), bf16),
            pltpu.VMEM((B, N_DEV, BDR, GDIM), bf16),
            pltpu.VMEM((B, N_DEV, BDR, GDIM), bf16),
            pltpu.SemaphoreType.DMA((N_DEV,)),
            pltpu.SemaphoreType.DMA((N_DEV,)),
            pltpu.SemaphoreType.DMA((N_DEV,)),
            pltpu.SemaphoreType.DMA((N_DEV,)),
        ],
        compiler_params=pltpu.CompilerParams(collective_id=0),
    )(x, Wq, K_ext, V_ext, Wo)


# device time: 23515 ns/iter; 1.0377x vs baseline; 1.0340x over previous
import os

import jax
import jax.numpy as jnp
from jax import lax
from jax.experimental import pallas as pl
from jax.experimental.pallas import tpu as pltpu

N_DEV = 4
B = 2
SQ_LOC = 128
D_MODEL = 512
HQ = 16
HQ_LOC = 4
DH = 64
SKV = 128
BLK = 64
GDIM = HQ_LOC * DH
BDR = HQ_LOC * SKV


def kernel(x, Wq, K_ext, V_ext, Wo):
    bf16 = jnp.bfloat16
    f32 = jnp.float32

    def body(x_ref, wq_ref, k_ref, v_ref, wo_ref, out_ref,
             wq_bf, wo_bf, wq_slots, wo_slots, k_bd, v_bd,
             wq_send, wq_recv, wo_send, wo_recv):
        my = lax.axis_index("i")
        left = lax.rem(my + N_DEV - 1, N_DEV)
        right = lax.rem(my + 1, N_DEV)
        opp = lax.rem(my + 2, N_DEV)

        barrier_sem = pltpu.get_barrier_semaphore()
        for nbr in (left, right, opp):
            pl.semaphore_signal(
                barrier_sem, inc=1,
                device_id=(nbr,), device_id_type=pl.DeviceIdType.MESH,
            )
        wq_bf[...] = wq_ref[...].astype(bf16)
        wo_bf[...] = wo_ref[...].astype(bf16)
        pl.semaphore_wait(barrier_sem, N_DEV - 1)

        txs = []

        def push(src, slots, ssem, rsem, dest):
            tx = pltpu.make_async_remote_copy(
                src_ref=src, dst_ref=slots.at[my],
                send_sem=ssem.at[dest], recv_sem=rsem.at[my],
                device_id=(dest,), device_id_type=pl.DeviceIdType.MESH,
            )
            tx.start()
            txs.append(tx)

        def wait_recv_slot(slots, rsem, origin):
            rx = pltpu.make_async_remote_copy(
                src_ref=slots.at[origin], dst_ref=slots.at[origin],
                send_sem=wq_send.at[origin], recv_sem=rsem.at[origin],
                device_id=(origin,), device_id_type=pl.DeviceIdType.MESH,
            )
            rx.wait_recv()

        skip_comm = bool(os.environ.get("SKIP_COMM"))
        if not skip_comm:
            for dest in (opp, right, left):
                push(wq_bf, wq_slots, wq_send, wq_recv, dest)
            for dest in (opp, right, left):
                push(wo_bf, wo_slots, wo_send, wo_recv, dest)


        k_bd[...] = jnp.zeros((B, N_DEV, BDR, GDIM), bf16)
        v_bd[...] = jnp.zeros((B, N_DEV, BDR, GDIM), bf16)
        for b in range(B):
            for g in range(N_DEV):
                for hh in range(HQ_LOC):
                    head = g * HQ_LOC + hh
                    r0, c0 = hh * SKV, hh * DH
                    k_bd[b, g, r0:r0 + SKV, c0:c0 + DH] = (
                        k_ref[b, :, head, :].astype(bf16))
                    v_bd[b, g, r0:r0 + SKV, c0:c0 + DH] = (
                        v_ref[b, :, head, :].astype(bf16))

        x2 = (x_ref[...].reshape(B * SQ_LOC, D_MODEL) * 0.125).astype(bf16)

        qi = lax.broadcasted_iota(jnp.int32, (SQ_LOC, HQ_LOC * SKV), 0)
        kj = lax.broadcasted_iota(jnp.int32, (SQ_LOC, HQ_LOC * SKV), 1)
        qb = my * (SQ_LOC // BLK) + qi // BLK
        kb = lax.rem(kj, SKV) // BLK
        mask = (qb == kb) | (kb == 0) | (lax.rem(qb + kb, 3) == 0)
        bias = jnp.where(mask, 0.0, -1e9).astype(f32)

        def attn_group(g, wq_g):
            q_g = jax.lax.dot_general(
                x2, wq_g, (((1,), (0,)), ((), ())),
                preferred_element_type=f32,
            ).astype(bf16)
            ctxs = []
            for b in range(B):
                q_b = q_g[b * SQ_LOC:(b + 1) * SQ_LOC, :]
                s = jax.lax.dot_general(
                    q_b, k_bd[b, g], (((1,), (1,)), ((), ())),
                    preferred_element_type=f32,
                )
                w = jnp.exp(s + bias)
                w3 = w.reshape(SQ_LOC, HQ_LOC, SKV)
                w3 = w3 / jnp.sum(w3, axis=-1, keepdims=True)
                w2 = w3.reshape(SQ_LOC, HQ_LOC * SKV).astype(bf16)
                ctxs.append(jax.lax.dot_general(
                    w2, v_bd[b, g], (((1,), (0,)), ((), ())),
                    preferred_element_type=f32,
                ).astype(bf16))
            return jnp.concatenate(ctxs, axis=0)

        def out_gemm(ctx, wo_g, acc):
            return acc + jax.lax.dot_general(
                ctx, wo_g, (((1,), (0,)), ((), ())),
                preferred_element_type=f32,
            )

        acc = jnp.zeros((B * SQ_LOC, D_MODEL), dtype=f32)
        if skip_comm:
            for g in range(N_DEV):
                acc = out_gemm(attn_group(g, wq_bf[...]), wo_bf[...], acc)
        else:
            acc = out_gemm(attn_group(my, wq_bf[...]), wo_bf[...], acc)
            for origin in (left, right, opp):
                wait_recv_slot(wq_slots, wq_recv, origin)
                ctx = attn_group(origin, wq_slots[origin])
                wait_recv_slot(wo_slots, wo_recv, origin)
                acc = out_gemm(ctx, wo_slots[origin], acc)
            for tx in txs:
                tx.wait_send()

        out_ref[...] = acc.reshape(B, SQ_LOC, D_MODEL)

    return pl.pallas_call(
        body,
        out_shape=jax.ShapeDtypeStruct((B, SQ_LOC, D_MODEL), jnp.float32),
        in_specs=[pl.BlockSpec(memory_space=pltpu.VMEM)] * 5,
        out_specs=pl.BlockSpec(memory_space=pltpu.VMEM),
        scratch_shapes=[
            pltpu.VMEM((D_MODEL, GDIM), bf16),
            pltpu.VMEM((GDIM, D_MODEL), bf16),
            pltpu.VMEM((N_DEV, D_MODEL, GDIM), bf16),
            pltpu.VMEM((N_DEV, GDIM, D_MODEL), bf16),
            pltpu.VMEM((B, N_DEV, BDR, GDIM), bf16),
            pltpu.VMEM((B, N_DEV, BDR, GDIM), bf16),
            pltpu.SemaphoreType.DMA((N_DEV,)),
            pltpu.SemaphoreType.DMA((N_DEV,)),
            pltpu.SemaphoreType.DMA((N_DEV,)),
            pltpu.SemaphoreType.DMA((N_DEV,)),
        ],
        compiler_params=pltpu.CompilerParams(collective_id=0),
    )(x, Wq, K_ext, V_ext, Wo)
